# Initial kernel scaffold; baseline (speedup 1.0000x reference)
import jax
import jax.numpy as jnp
from jax.experimental import pallas as pl


def kernel(x, x_new, importance, memory_bank):
    raise NotImplementedError("write your pallas kernel here")


# speedup vs baseline: 1.7477x; 1.7477x over previous
import functools

import jax
import jax.numpy as jnp
from jax import lax
from jax.experimental import pallas as pl
from jax.experimental.pallas import tpu as pltpu

_RETENTION = 0.9
_SPEED = 5.0
_BS = 2048


def _fused_body(x_ref, imp_ref, m_ref, xnew_ref, out_ref, new_ref,
                acc_ref, sumw_ref, nsteps):
    i = pl.program_id(0)
    m = m_ref[...]
    x = x_ref[...]

    imp_t = imp_ref[...]
    oneh = (lax.broadcasted_iota(jnp.int32, (1, imp_t.shape[1]), 1)
            == i).astype(jnp.float32)
    imp_col = jnp.sum(imp_t * oneh, axis=1, keepdims=True)
    mask = imp_col > (1.0 - _RETENTION)
    new_ref[...] = jnp.where(mask, xnew_ref[...], m)

    num = lax.dot_general(x, m, (((1,), (1,)), ((), ())),
                          preferred_element_type=jnp.float32)
    x_norm = jnp.sqrt(jnp.sum(x * x, axis=1, keepdims=True))
    m_norm = jnp.sqrt(jnp.sum(m * m, axis=1)).reshape(1, -1)
    denom = jnp.maximum(x_norm * m_norm, 1e-8)
    w = jnp.exp((_SPEED / 1.0) * (num / denom))

    part = lax.dot_general(w, m, (((1,), (0,)), ((), ())),
                           preferred_element_type=jnp.float32)
    wsum = jnp.sum(w, axis=1, keepdims=True)

    @pl.when(i == 0)
    def _init():
        acc_ref[...] = part
        sumw_ref[...] = wsum

    @pl.when(i > 0)
    def _accum():
        acc_ref[...] += part
        sumw_ref[...] += wsum

    @pl.when(i == nsteps - 1)
    def _final():
        out_ref[...] = acc_ref[...] / sumw_ref[...]


def kernel(x, x_new, importance, memory_bank):
    size, dim = memory_bank.shape
    b = x.shape[0]
    bs = _BS if size % _BS == 0 else size
    nsteps = size // bs
    w = -(-nsteps // 128) * 128
    imp_t = importance.reshape(nsteps, bs).T
    if w != nsteps:
        imp_t = jnp.pad(imp_t, ((0, 0), (0, w - nsteps)))

    out, new_bank = pl.pallas_call(
        functools.partial(_fused_body, nsteps=nsteps),
        grid=(nsteps,),
        in_specs=[
            pl.BlockSpec((b, dim), lambda i: (0, 0)),
            pl.BlockSpec((bs, w), lambda i: (0, 0)),
            pl.BlockSpec((bs, dim), lambda i: (i, 0)),
            pl.BlockSpec((bs, dim), lambda i: (i, 0)),
        ],
        out_specs=[
            pl.BlockSpec((b, dim), lambda i: (0, 0)),
            pl.BlockSpec((bs, dim), lambda i: (i, 0)),
        ],
        out_shape=[
            jax.ShapeDtypeStruct((b, dim), jnp.float32),
            jax.ShapeDtypeStruct((size, dim), jnp.float32),
        ],
        scratch_shapes=[
            pltpu.VMEM((b, dim), jnp.float32),
            pltpu.VMEM((b, 1), jnp.float32),
        ],
    )(x, imp_t, memory_bank, x_new)
    return out, new_bank
